# Initial kernel scaffold; baseline (speedup 1.0000x reference)
import functools

import jax
import jax.numpy as jnp
from jax import lax
from jax.experimental import pallas as pl
from jax.experimental.pallas import tpu as pltpu
from jax.experimental.pallas import tpu_sc as plsc

N = 10000
E = 320000
NPAD = 10016
ROWS_PT = NPAD // 16
NC = 2
NS = 16
NW = NC * NS
EPW = E // NW
CHUNK = 80
NCH = EPW // CHUNK

_MESH = plsc.VectorSubcoreMesh(core_axis_name="c", subcore_axis_name="s")


def _deg_body(dst_hbm, ones_hbm, zeros_hbm, out_hbm, idx_v, ones_v, acc_sh, sem):
    cid = lax.axis_index("c")
    sid = lax.axis_index("s")
    wid = cid * NS + sid
    row0 = sid * ROWS_PT
    pltpu.sync_copy(zeros_hbm.at[pl.ds(row0, ROWS_PT)], acc_sh.at[pl.ds(row0, ROWS_PT)])
    pltpu.sync_copy(ones_hbm, ones_v)
    pltpu.sync_copy(dst_hbm.at[wid], idx_v)
    plsc.subcore_barrier()

    def body(j, carry):
        pltpu.sync_copy(ones_v, acc_sh.at[idx_v.at[j]], add=True)
        return carry

    lax.fori_loop(0, NCH, body, 0)
    plsc.subcore_barrier()
    pltpu.sync_copy(acc_sh.at[pl.ds(row0, ROWS_PT)], out_hbm.at[cid, pl.ds(row0, ROWS_PT)])


_deg_call = functools.partial(
    pl.kernel,
    out_type=jax.ShapeDtypeStruct((NC, NPAD, 8), jnp.float32),
    mesh=_MESH,
    scratch_types=[
        pltpu.VMEM((NCH, CHUNK), jnp.int32),
        pltpu.VMEM((CHUNK, 8), jnp.float32),
        pltpu.VMEM_SHARED((NPAD, 8), jnp.float32),
        pltpu.SemaphoreType.DMA,
    ],
)(_deg_body)


def _agg_body(y_hbm, src_hbm, dst_hbm, zeros_hbm, out_hbm,
              sidx_v, didx_v, rows_v, acc_sh, sem):
    cid = lax.axis_index("c")
    sid = lax.axis_index("s")
    wid = cid * NS + sid
    row0 = sid * ROWS_PT
    pltpu.sync_copy(zeros_hbm.at[pl.ds(row0, ROWS_PT)], acc_sh.at[pl.ds(row0, ROWS_PT)])
    pltpu.sync_copy(src_hbm.at[wid], sidx_v)
    pltpu.sync_copy(dst_hbm.at[wid], didx_v)
    plsc.subcore_barrier()

    def body(j, carry):
        pltpu.async_copy(y_hbm.at[sidx_v.at[j]], rows_v, sem).wait()
        pltpu.sync_copy(rows_v, acc_sh.at[didx_v.at[j]], add=True)
        return carry

    lax.fori_loop(0, NCH, body, 0)
    plsc.subcore_barrier()
    pltpu.sync_copy(acc_sh.at[pl.ds(row0, ROWS_PT)], out_hbm.at[cid, pl.ds(row0, ROWS_PT)])


_agg_call = functools.partial(
    pl.kernel,
    out_type=jax.ShapeDtypeStruct((NC, NPAD, 32), jnp.float32),
    mesh=_MESH,
    scratch_types=[
        pltpu.VMEM((NCH, CHUNK), jnp.int32),
        pltpu.VMEM((NCH, CHUNK), jnp.int32),
        pltpu.VMEM((CHUNK, 32), jnp.float32),
        pltpu.VMEM_SHARED((NPAD, 32), jnp.float32),
        pltpu.SemaphoreType.DMA,
    ],
)(_agg_body)


def _mm1_body(x_ref, w_ref, degp_ref, y_ref, dinv_ref):
    deg8 = degp_ref[0] + degp_ref[1]
    dinv8 = jnp.where(deg8 > 0.0, lax.rsqrt(jnp.maximum(deg8, 1e-12)), 0.0)
    dinv32 = jnp.concatenate([dinv8, dinv8, dinv8, dinv8], axis=1)
    y_ref[...] = jnp.dot(x_ref[...], w_ref[...],
                         preferred_element_type=jnp.float32) * dinv32
    dinv_ref[...] = dinv32


_mm1_call = pl.pallas_call(
    _mm1_body,
    out_shape=(
        jax.ShapeDtypeStruct((NPAD, 32), jnp.float32),
        jax.ShapeDtypeStruct((NPAD, 32), jnp.float32),
    ),
)


def _mid_body(s1p_ref, dinv_ref, b1_ref, y2_ref):
    s = s1p_ref[0] + s1p_ref[1]
    h = jnp.maximum(dinv_ref[...] * s + b1_ref[...], 0.0)
    y2_ref[...] = dinv_ref[...] * h


_mid_call = pl.pallas_call(
    _mid_body,
    out_shape=jax.ShapeDtypeStruct((NPAD, 32), jnp.float32),
)


def _out_body(s2p_ref, dinv_ref, wmu_ref, bmu_ref, wls_ref, bls_ref,
              mu_ref, ls_ref):
    agg = dinv_ref[...] * (s2p_ref[0] + s2p_ref[1])
    mu_ref[...] = jnp.dot(agg, wmu_ref[...],
                          preferred_element_type=jnp.float32) + bmu_ref[...]
    ls_ref[...] = jnp.dot(agg, wls_ref[...],
                          preferred_element_type=jnp.float32) + bls_ref[...]


_out_call = pl.pallas_call(
    _out_body,
    out_shape=(
        jax.ShapeDtypeStruct((NPAD, 16), jnp.float32),
        jax.ShapeDtypeStruct((NPAD, 16), jnp.float32),
    ),
)


def kernel(x, edge_index, W1, b1, Wmu, bmu, Wls, bls):
    src = edge_index[0].astype(jnp.int32).reshape(NW, NCH, CHUNK)
    dst = edge_index[1].astype(jnp.int32).reshape(NW, NCH, CHUNK)
    xp = jnp.pad(x, ((0, NPAD - N), (0, 0)))
    zeros32 = jnp.zeros((NPAD, 32), jnp.float32)
    zeros8 = jnp.zeros((NPAD, 8), jnp.float32)
    ones8 = jnp.ones((CHUNK, 8), jnp.float32)

    degp = _deg_call(dst, ones8, zeros8)
    y1, dinv32 = _mm1_call(xp, W1, degp)
    s1p = _agg_call(y1, src, dst, zeros32)
    y2 = _mid_call(s1p, dinv32, b1.reshape(1, 32))
    s2p = _agg_call(y2, src, dst, zeros32)
    mu, ls = _out_call(s2p, dinv32, Wmu, bmu.reshape(1, 16),
                       Wls, bls.reshape(1, 16))
    return (mu[:N], ls[:N])


# speedup vs baseline: 26.9816x; 26.9816x over previous
import functools

import jax
import jax.numpy as jnp
from jax import lax
from jax.experimental import pallas as pl
from jax.experimental.pallas import tpu as pltpu
from jax.experimental.pallas import tpu_sc as plsc

N = 10000
E = 320000
NPAD = 10112
ROWS_PT = NPAD // 16
NC = 2
NS = 16
NW = NC * NS
EPW = E // NW
CHUNK = 80
NCH = EPW // CHUNK

_MESH = plsc.VectorSubcoreMesh(core_axis_name="c", subcore_axis_name="s")
_SC_PARAMS = pltpu.CompilerParams(use_tc_tiling_on_sc=False)


def _deg_body(dst_hbm, ones_hbm, zeros_hbm, out_hbm, idx_v, ones_v, acc_sh, sem):
    cid = lax.axis_index("c")
    sid = lax.axis_index("s")
    wid = cid * NS + sid
    row0 = sid * ROWS_PT
    pltpu.sync_copy(zeros_hbm.at[pl.ds(row0, ROWS_PT)], acc_sh.at[pl.ds(row0, ROWS_PT)])
    pltpu.sync_copy(ones_hbm, ones_v)
    pltpu.sync_copy(dst_hbm.at[wid], idx_v)
    plsc.subcore_barrier()

    def body(j, carry):
        pltpu.sync_copy(ones_v, acc_sh.at[idx_v.at[j]], add=True)
        return carry

    lax.fori_loop(0, NCH, body, 0)
    plsc.subcore_barrier()
    pltpu.sync_copy(acc_sh.at[pl.ds(row0, ROWS_PT)], out_hbm.at[cid, pl.ds(row0, ROWS_PT)])


_deg_call = functools.partial(
    pl.kernel,
    out_type=jax.ShapeDtypeStruct((NC, NPAD, 8), jnp.float32),
    mesh=_MESH,
    compiler_params=_SC_PARAMS,
    scratch_types=[
        pltpu.VMEM((NCH, CHUNK), jnp.int32),
        pltpu.VMEM((CHUNK, 8), jnp.float32),
        pltpu.VMEM_SHARED((NPAD, 8), jnp.float32),
        pltpu.SemaphoreType.DMA,
    ],
)(_deg_body)


def _agg_body(y_hbm, src_hbm, dst_hbm, zeros_hbm, out_hbm,
              sidx_v, didx_v, rows_v, acc_sh, sem):
    cid = lax.axis_index("c")
    sid = lax.axis_index("s")
    wid = cid * NS + sid
    row0 = sid * ROWS_PT
    pltpu.sync_copy(zeros_hbm.at[pl.ds(row0, ROWS_PT)], acc_sh.at[pl.ds(row0, ROWS_PT)])
    pltpu.sync_copy(src_hbm.at[wid], sidx_v)
    pltpu.sync_copy(dst_hbm.at[wid], didx_v)
    plsc.subcore_barrier()

    def body(j, carry):
        pltpu.async_copy(y_hbm.at[sidx_v.at[j]], rows_v, sem).wait()
        pltpu.sync_copy(rows_v, acc_sh.at[didx_v.at[j]], add=True)
        return carry

    lax.fori_loop(0, NCH, body, 0)
    plsc.subcore_barrier()
    pltpu.sync_copy(acc_sh.at[pl.ds(row0, ROWS_PT)], out_hbm.at[cid, pl.ds(row0, ROWS_PT)])


_agg_call = functools.partial(
    pl.kernel,
    out_type=jax.ShapeDtypeStruct((NC, NPAD, 32), jnp.float32),
    mesh=_MESH,
    compiler_params=_SC_PARAMS,
    scratch_types=[
        pltpu.VMEM((NCH, CHUNK), jnp.int32),
        pltpu.VMEM((NCH, CHUNK), jnp.int32),
        pltpu.VMEM((CHUNK, 32), jnp.float32),
        pltpu.VMEM_SHARED((NPAD, 32), jnp.float32),
        pltpu.SemaphoreType.DMA,
    ],
)(_agg_body)


def _mm1_body(x_ref, w_ref, degp_ref, y_ref, dinv_ref):
    deg8 = degp_ref[0] + degp_ref[1]
    dinv8 = jnp.where(deg8 > 0.0, lax.rsqrt(jnp.maximum(deg8, 1e-12)), 0.0)
    dinv32 = jnp.concatenate([dinv8, dinv8, dinv8, dinv8], axis=1)
    y_ref[...] = jnp.dot(x_ref[...], w_ref[...],
                         preferred_element_type=jnp.float32) * dinv32
    dinv_ref[...] = dinv32


_mm1_call = pl.pallas_call(
    _mm1_body,
    out_shape=(
        jax.ShapeDtypeStruct((NPAD, 32), jnp.float32),
        jax.ShapeDtypeStruct((NPAD, 32), jnp.float32),
    ),
)


def _mid_body(s1p_ref, dinv_ref, b1_ref, y2_ref):
    s = s1p_ref[0] + s1p_ref[1]
    h = jnp.maximum(dinv_ref[...] * s + b1_ref[...], 0.0)
    y2_ref[...] = dinv_ref[...] * h


_mid_call = pl.pallas_call(
    _mid_body,
    out_shape=jax.ShapeDtypeStruct((NPAD, 32), jnp.float32),
)


def _out_body(s2p_ref, dinv_ref, wmu_ref, bmu_ref, wls_ref, bls_ref,
              mu_ref, ls_ref):
    agg = dinv_ref[...] * (s2p_ref[0] + s2p_ref[1])
    mu_ref[...] = jnp.dot(agg, wmu_ref[...],
                          preferred_element_type=jnp.float32) + bmu_ref[...]
    ls_ref[...] = jnp.dot(agg, wls_ref[...],
                          preferred_element_type=jnp.float32) + bls_ref[...]


_out_call = pl.pallas_call(
    _out_body,
    out_shape=(
        jax.ShapeDtypeStruct((NPAD, 16), jnp.float32),
        jax.ShapeDtypeStruct((NPAD, 16), jnp.float32),
    ),
)


def kernel(x, edge_index, W1, b1, Wmu, bmu, Wls, bls):
    src = edge_index[0].astype(jnp.int32).reshape(NW, NCH, CHUNK)
    dst = edge_index[1].astype(jnp.int32).reshape(NW, NCH, CHUNK)
    xp = jnp.pad(x, ((0, NPAD - N), (0, 0)))
    zeros32 = jnp.zeros((NPAD, 32), jnp.float32)
    zeros8 = jnp.zeros((NPAD, 8), jnp.float32)
    ones8 = jnp.ones((CHUNK, 8), jnp.float32)

    degp = _deg_call(dst, ones8, zeros8)
    y1, dinv32 = _mm1_call(xp, W1, degp)
    s1p = _agg_call(y1, src, dst, zeros32)
    y2 = _mid_call(s1p, dinv32, b1.reshape(1, 32))
    s2p = _agg_call(y2, src, dst, zeros32)
    mu, ls = _out_call(s2p, dinv32, Wmu, bmu.reshape(1, 16),
                       Wls, bls.reshape(1, 16))
    return (mu[:N], ls[:N])


# speedup vs baseline: 47.0614x; 1.7442x over previous
import functools

import jax
import jax.numpy as jnp
from jax import lax
from jax.experimental import pallas as pl
from jax.experimental.pallas import tpu as pltpu
from jax.experimental.pallas import tpu_sc as plsc

N = 10000
E = 320000
NPAD = 10112
ROWS_PT = NPAD // 16
NC = 2
NS = 16
NW = NC * NS
EPW = E // NW
CHUNK = 80
NCH = EPW // CHUNK
NBUF = 4

_MESH = plsc.VectorSubcoreMesh(core_axis_name="c", subcore_axis_name="s")
_SC_PARAMS = pltpu.CompilerParams(use_tc_tiling_on_sc=False)


def _deg_body(dst_hbm, ones_hbm, zeros_hbm, out_hbm, idx_v, ones_v, acc_sh, sem):
    cid = lax.axis_index("c")
    sid = lax.axis_index("s")
    wid = cid * NS + sid
    row0 = sid * ROWS_PT
    pltpu.sync_copy(zeros_hbm.at[pl.ds(row0, ROWS_PT)], acc_sh.at[pl.ds(row0, ROWS_PT)])
    pltpu.sync_copy(ones_hbm, ones_v)
    pltpu.sync_copy(dst_hbm.at[wid], idx_v)
    plsc.subcore_barrier()

    def body(j, carry):
        pltpu.sync_copy(ones_v, acc_sh.at[idx_v.at[j]], add=True)
        return carry

    lax.fori_loop(0, NCH, body, 0)
    plsc.subcore_barrier()
    pltpu.sync_copy(acc_sh.at[pl.ds(row0, ROWS_PT)], out_hbm.at[cid, pl.ds(row0, ROWS_PT)])


_deg_call = functools.partial(
    pl.kernel,
    out_type=jax.ShapeDtypeStruct((NC, NPAD, 8), jnp.float32),
    mesh=_MESH,
    compiler_params=_SC_PARAMS,
    scratch_types=[
        pltpu.VMEM((NCH, CHUNK), jnp.int32),
        pltpu.VMEM((CHUNK, 8), jnp.float32),
        pltpu.VMEM_SHARED((NPAD, 8), jnp.float32),
        pltpu.SemaphoreType.DMA,
    ],
)(_deg_body)


def _agg_body(y_hbm, src_hbm, dst_hbm, zeros_hbm, out_hbm,
              sidx_v, didx_v, rows_v, acc_sh, gsem, ssem):
    cid = lax.axis_index("c")
    sid = lax.axis_index("s")
    wid = cid * NS + sid
    row0 = sid * ROWS_PT
    pltpu.sync_copy(zeros_hbm.at[pl.ds(row0, ROWS_PT)], acc_sh.at[pl.ds(row0, ROWS_PT)])
    pltpu.sync_copy(src_hbm.at[wid], sidx_v)
    pltpu.sync_copy(dst_hbm.at[wid], didx_v)
    plsc.subcore_barrier()

    for b in range(NBUF - 1):
        pltpu.async_copy(y_hbm.at[sidx_v.at[b]], rows_v.at[b], gsem.at[b])

    def body(j, carry):
        b = lax.rem(j, NBUF)
        pltpu.make_async_copy(y_hbm.at[sidx_v.at[j]], rows_v.at[b], gsem.at[b]).wait()
        pltpu.async_copy(rows_v.at[b], acc_sh.at[didx_v.at[j]], ssem.at[b], add=True)
        g = j + NBUF - 1
        bg = lax.rem(g, NBUF)

        @pl.when(jnp.logical_and(j >= 1, g < NCH))
        def _wait_prev_scatter():
            pltpu.make_async_copy(rows_v.at[bg], acc_sh.at[didx_v.at[0]], ssem.at[bg]).wait()

        @pl.when(g < NCH)
        def _issue_gather():
            pltpu.async_copy(y_hbm.at[sidx_v.at[g]], rows_v.at[bg], gsem.at[bg])

        return carry

    lax.fori_loop(0, NCH, body, 0)
    for k in range(NBUF):
        b = (NCH - NBUF + k) % NBUF
        pltpu.make_async_copy(rows_v.at[b], acc_sh.at[didx_v.at[0]], ssem.at[b]).wait()
    plsc.subcore_barrier()
    pltpu.sync_copy(acc_sh.at[pl.ds(row0, ROWS_PT)], out_hbm.at[cid, pl.ds(row0, ROWS_PT)])


_agg_call = functools.partial(
    pl.kernel,
    out_type=jax.ShapeDtypeStruct((NC, NPAD, 32), jnp.float32),
    mesh=_MESH,
    compiler_params=_SC_PARAMS,
    scratch_types=[
        pltpu.VMEM((NCH, CHUNK), jnp.int32),
        pltpu.VMEM((NCH, CHUNK), jnp.int32),
        pltpu.VMEM((NBUF, CHUNK, 32), jnp.float32),
        pltpu.VMEM_SHARED((NPAD, 32), jnp.float32),
        pltpu.SemaphoreType.DMA((NBUF,)),
        pltpu.SemaphoreType.DMA((NBUF,)),
    ],
)(_agg_body)


def _mm1_body(x_ref, w_ref, degp_ref, y_ref, dinv_ref):
    deg8 = degp_ref[0] + degp_ref[1]
    dinv8 = jnp.where(deg8 > 0.0, lax.rsqrt(jnp.maximum(deg8, 1e-12)), 0.0)
    dinv32 = jnp.concatenate([dinv8, dinv8, dinv8, dinv8], axis=1)
    dinv_ref[...] = dinv32
    y_ref[...] = jnp.dot(x_ref[...], w_ref[...],
                         preferred_element_type=jnp.float32) * dinv32[:N]


_mm1_call = pl.pallas_call(
    _mm1_body,
    out_shape=(
        jax.ShapeDtypeStruct((N, 32), jnp.float32),
        jax.ShapeDtypeStruct((NPAD, 32), jnp.float32),
    ),
)


def _mid_body(s1p_ref, dinv_ref, b1_ref, y2_ref):
    s = s1p_ref[0, :N] + s1p_ref[1, :N]
    dinv = dinv_ref[:N]
    h = jnp.maximum(dinv * s + b1_ref[...], 0.0)
    y2_ref[...] = dinv * h


_mid_call = pl.pallas_call(
    _mid_body,
    out_shape=jax.ShapeDtypeStruct((N, 32), jnp.float32),
)


def _out_body(s2p_ref, dinv_ref, wmu_ref, bmu_ref, wls_ref, bls_ref,
              mu_ref, ls_ref):
    agg = dinv_ref[:N] * (s2p_ref[0, :N] + s2p_ref[1, :N])
    mu_ref[...] = jnp.dot(agg, wmu_ref[...],
                          preferred_element_type=jnp.float32) + bmu_ref[...]
    ls_ref[...] = jnp.dot(agg, wls_ref[...],
                          preferred_element_type=jnp.float32) + bls_ref[...]


_out_call = pl.pallas_call(
    _out_body,
    out_shape=(
        jax.ShapeDtypeStruct((N, 16), jnp.float32),
        jax.ShapeDtypeStruct((N, 16), jnp.float32),
    ),
)


def kernel(x, edge_index, W1, b1, Wmu, bmu, Wls, bls):
    src = edge_index[0].astype(jnp.int32).reshape(NW, NCH, CHUNK)
    dst = edge_index[1].astype(jnp.int32).reshape(NW, NCH, CHUNK)
    zeros32 = jnp.zeros((NPAD, 32), jnp.float32)
    zeros8 = jnp.zeros((NPAD, 8), jnp.float32)
    ones8 = jnp.ones((CHUNK, 8), jnp.float32)

    degp = _deg_call(dst, ones8, zeros8)
    y1, dinv32 = _mm1_call(x, W1, degp)
    s1p = _agg_call(y1, src, dst, zeros32)
    y2 = _mid_call(s1p, dinv32, b1.reshape(1, 32))
    s2p = _agg_call(y2, src, dst, zeros32)
    mu, ls = _out_call(s2p, dinv32, Wmu, bmu.reshape(1, 16),
                       Wls, bls.reshape(1, 16))
    return (mu, ls)


# speedup vs baseline: 49.2559x; 1.0466x over previous
import functools

import jax
import jax.numpy as jnp
from jax import lax
from jax.experimental import pallas as pl
from jax.experimental.pallas import tpu as pltpu
from jax.experimental.pallas import tpu_sc as plsc

N = 10000
E = 320000
NPAD = 10240
ROWS_PT = NPAD // 16
NGRP = ROWS_PT // 16
NC = 2
NS = 16
NW = NC * NS
EPW = E // NW
CHUNK = 80
NCH = EPW // CHUNK
HROWS = ROWS_PT // 2
NBUF = 4

_MESH = plsc.VectorSubcoreMesh(core_axis_name="c", subcore_axis_name="s")
_SC_PARAMS = pltpu.CompilerParams(use_tc_tiling_on_sc=False,
                                  needs_layout_passes=False)


def _fast_rsqrt(x):
    i = plsc.bitcast(x, jnp.int32)
    i = 0x5F3759DF - lax.shift_right_logical(i, 1)
    y = plsc.bitcast(i, jnp.float32)
    xh = 0.5 * x
    for _ in range(3):
        y = y * (1.5 - xh * y * y)
    return jnp.where(x > 0.0, y, 0.0)


def _zero_acc(zeros_hbm, acc_sh, row0):
    pltpu.sync_copy(zeros_hbm.at[pl.ds(row0, ROWS_PT)],
                    acc_sh.at[pl.ds(row0, ROWS_PT)])


def _agg_pipeline(table_hbm, sidx_v, didx_v, dbase, rows_v, acc_sh, gsem, ssem):
    for b in range(NBUF - 1):
        pltpu.async_copy(table_hbm.at[sidx_v.at[b]], rows_v.at[b], gsem.at[b])

    def body(j, carry):
        b = lax.rem(j, NBUF)
        pltpu.make_async_copy(table_hbm.at[sidx_v.at[j]], rows_v.at[b],
                              gsem.at[b]).wait()
        pltpu.async_copy(rows_v.at[b], acc_sh.at[didx_v.at[dbase + j]],
                         ssem.at[b], add=True)
        g = j + NBUF - 1
        bg = lax.rem(g, NBUF)

        @pl.when(jnp.logical_and(j >= 1, g < NCH))
        def _wait_prev_scatter():
            pltpu.make_async_copy(rows_v.at[bg], acc_sh.at[didx_v.at[dbase]],
                                  ssem.at[bg]).wait()

        @pl.when(g < NCH)
        def _issue_gather():
            pltpu.async_copy(table_hbm.at[sidx_v.at[g]], rows_v.at[bg],
                             gsem.at[bg])

        return carry

    lax.fori_loop(0, NCH, body, 0)
    for k in range(NBUF):
        b = (NCH - NBUF + k) % NBUF
        pltpu.make_async_copy(rows_v.at[b], acc_sh.at[didx_v.at[dbase]],
                              ssem.at[b]).wait()


def _sca_body(z_hbm, src_hbm, dst_hbm, zeros_hbm,
              s1_out, y1_out, dinv_out,
              sidx_v, didx_v, hist_v, red_v, zrow_v, dinv32_v, dinv_v,
              rows_v, hist_sh, acc_sh, gsem, ssem):
    cid = lax.axis_index("c")
    sid = lax.axis_index("s")
    wid = 2 * sid + cid
    row0 = sid * ROWS_PT

    def zero_hist(i, c):
        hist_v[pl.ds(i * 16, 16)] = jnp.zeros((16,), jnp.float32)
        return c

    lax.fori_loop(0, NPAD // 16, zero_hist, 0)
    ones16 = jnp.ones((16,), jnp.float32)

    def hist_row(r, c):
        for k in range(CHUNK // 16):
            idx16 = didx_v[r, pl.ds(k * 16, 16)]
            plsc.addupdate_scatter(hist_v, [idx16], ones16)
        return c

    for half in range(2):
        pltpu.sync_copy(dst_hbm.at[2 * sid + half], didx_v)
        lax.fori_loop(0, NCH, hist_row, 0)
    pltpu.sync_copy(hist_v, hist_sh.at[sid])
    plsc.subcore_barrier()
    for k in range(NS):
        pltpu.sync_copy(hist_sh.at[k, pl.ds(row0, ROWS_PT)], red_v.at[k])

    def reduce_grp(g, c):
        acc = jnp.zeros((16,), jnp.float32)
        for k in range(NS):
            acc = acc + red_v[k, pl.ds(g * 16, 16)]
        dinv_v[pl.ds(g * 16, 16)] = _fast_rsqrt(acc)
        return c

    lax.fori_loop(0, NGRP, reduce_grp, 0)

    for half in range(2):
        hbase = half * HROWS

        def scale_row(r, c):
            idxr = jnp.broadcast_to(hbase + r, (16,)).astype(jnp.int32)
            dv = plsc.load_gather(dinv_v, [idxr])
            zrow_v[r, pl.ds(0, 16)] = zrow_v[r, pl.ds(0, 16)] * dv
            zrow_v[r, pl.ds(16, 16)] = zrow_v[r, pl.ds(16, 16)] * dv
            dinv32_v[r, pl.ds(0, 16)] = dv
            dinv32_v[r, pl.ds(16, 16)] = dv
            return c

        pltpu.sync_copy(z_hbm.at[pl.ds(row0 + hbase, HROWS)], zrow_v)
        lax.fori_loop(0, HROWS, scale_row, 0)
        pltpu.sync_copy(
            zrow_v, y1_out.at[pl.ds(cid * NPAD + row0 + hbase, HROWS)])

        @pl.when(cid == 0)
        def _dump_dinv():
            pltpu.sync_copy(dinv32_v, dinv_out.at[pl.ds(row0 + hbase, HROWS)])

    _zero_acc(zeros_hbm, acc_sh, row0)
    pltpu.sync_copy(src_hbm.at[wid], sidx_v)
    pltpu.sync_copy(dst_hbm.at[wid], didx_v)
    plsc.subcore_barrier()

    _agg_pipeline(y1_out, sidx_v, didx_v, 0, rows_v, acc_sh,
                  gsem, ssem)
    plsc.subcore_barrier()
    pltpu.sync_copy(acc_sh.at[pl.ds(row0, ROWS_PT)],
                    s1_out.at[cid, pl.ds(row0, ROWS_PT)])


_sca_call = functools.partial(
    pl.kernel,
    out_type=(
        jax.ShapeDtypeStruct((NC, NPAD, 32), jnp.float32),
        jax.ShapeDtypeStruct((NC * NPAD, 32), jnp.float32),
        jax.ShapeDtypeStruct((NPAD, 32), jnp.float32),
    ),
    mesh=_MESH,
    compiler_params=_SC_PARAMS,
    scratch_types=[
        pltpu.VMEM((NCH, CHUNK), jnp.int32),
        pltpu.VMEM((NCH, CHUNK), jnp.int32),
        pltpu.VMEM((NPAD,), jnp.float32),
        pltpu.VMEM((NS, ROWS_PT), jnp.float32),
        pltpu.VMEM((HROWS, 32), jnp.float32),
        pltpu.VMEM((HROWS, 32), jnp.float32),
        pltpu.VMEM((ROWS_PT,), jnp.float32),
        pltpu.VMEM((NBUF, CHUNK, 32), jnp.float32),
        pltpu.VMEM_SHARED((NS, NPAD), jnp.float32),
        pltpu.VMEM_SHARED((NPAD, 32), jnp.float32),
        pltpu.SemaphoreType.DMA((NBUF,)),
        pltpu.SemaphoreType.DMA((NBUF,)),
    ],
)(_sca_body)


def _scb_body(s1p_hbm, dinv_hbm, b1_hbm, src_hbm, dst_hbm, zeros_hbm,
              s2_out, y2_out,
              sidx_v, didx_v, s0row_v, s1row_v, dinv32_v, b1_v,
              rows_v, acc_sh, gsem, ssem):
    cid = lax.axis_index("c")
    sid = lax.axis_index("s")
    wid = 2 * sid + cid
    row0 = sid * ROWS_PT

    pltpu.sync_copy(b1_hbm, b1_v)

    def mid_row(r, c):
        for k in range(2):
            sl = pl.ds(k * 16, 16)
            s = s0row_v[r, sl] + s1row_v[r, sl]
            dv = dinv32_v[r, sl]
            h = jnp.maximum(dv * s + b1_v[sl], 0.0)
            s0row_v[r, sl] = dv * h
        return c

    for half in range(2):
        hbase = half * HROWS
        pltpu.sync_copy(s1p_hbm.at[0, pl.ds(row0 + hbase, HROWS)], s0row_v)
        pltpu.sync_copy(s1p_hbm.at[1, pl.ds(row0 + hbase, HROWS)], s1row_v)
        pltpu.sync_copy(dinv_hbm.at[pl.ds(row0 + hbase, HROWS)], dinv32_v)
        lax.fori_loop(0, HROWS, mid_row, 0)
        pltpu.sync_copy(
            s0row_v, y2_out.at[pl.ds(cid * NPAD + row0 + hbase, HROWS)])

    _zero_acc(zeros_hbm, acc_sh, row0)
    pltpu.sync_copy(src_hbm.at[wid], sidx_v)
    pltpu.sync_copy(dst_hbm.at[wid], didx_v)
    plsc.subcore_barrier()

    _agg_pipeline(y2_out, sidx_v, didx_v, 0, rows_v, acc_sh, gsem, ssem)
    plsc.subcore_barrier()
    pltpu.sync_copy(acc_sh.at[pl.ds(row0, ROWS_PT)],
                    s2_out.at[cid, pl.ds(row0, ROWS_PT)])


_scb_call = functools.partial(
    pl.kernel,
    out_type=(
        jax.ShapeDtypeStruct((NC, NPAD, 32), jnp.float32),
        jax.ShapeDtypeStruct((NC * NPAD, 32), jnp.float32),
    ),
    mesh=_MESH,
    compiler_params=_SC_PARAMS,
    scratch_types=[
        pltpu.VMEM((NCH, CHUNK), jnp.int32),
        pltpu.VMEM((NCH, CHUNK), jnp.int32),
        pltpu.VMEM((HROWS, 32), jnp.float32),
        pltpu.VMEM((HROWS, 32), jnp.float32),
        pltpu.VMEM((HROWS, 32), jnp.float32),
        pltpu.VMEM((32,), jnp.float32),
        pltpu.VMEM((NBUF, CHUNK, 32), jnp.float32),
        pltpu.VMEM_SHARED((NPAD, 32), jnp.float32),
        pltpu.SemaphoreType.DMA((NBUF,)),
        pltpu.SemaphoreType.DMA((NBUF,)),
    ],
)(_scb_body)


def _mm_body(x_ref, w_ref, z_ref):
    z = jnp.dot(x_ref[...], w_ref[...], preferred_element_type=jnp.float32)
    z_ref[...] = jnp.concatenate(
        [z, jnp.zeros((NPAD - N, 32), jnp.float32)], axis=0)


_mm_call = pl.pallas_call(
    _mm_body,
    out_shape=jax.ShapeDtypeStruct((NPAD, 32), jnp.float32),
)


def _out_body(s2p_ref, dinv_ref, wmu_ref, bmu_ref, wls_ref, bls_ref,
              mu_ref, ls_ref):
    agg = dinv_ref[:N] * (s2p_ref[0, :N] + s2p_ref[1, :N])
    mu_ref[...] = jnp.dot(agg, wmu_ref[...],
                          preferred_element_type=jnp.float32) + bmu_ref[...]
    ls_ref[...] = jnp.dot(agg, wls_ref[...],
                          preferred_element_type=jnp.float32) + bls_ref[...]


_out_call = pl.pallas_call(
    _out_body,
    out_shape=(
        jax.ShapeDtypeStruct((N, 16), jnp.float32),
        jax.ShapeDtypeStruct((N, 16), jnp.float32),
    ),
)


def kernel(x, edge_index, W1, b1, Wmu, bmu, Wls, bls):
    src = edge_index[0].astype(jnp.int32).reshape(NW, NCH, CHUNK)
    src = src + (jnp.arange(NW, dtype=jnp.int32)[:, None, None] & 1) * NPAD
    dst32 = edge_index[1].astype(jnp.int32).reshape(NW, NCH, CHUNK)
    zeros32 = jnp.zeros((NPAD, 32), jnp.float32)

    z = _mm_call(x, W1)
    s1p, _y1, dinv32 = _sca_call(z, src, dst32, zeros32)
    s2p, _y2 = _scb_call(s1p, dinv32, b1, src, dst32, zeros32)
    mu, ls = _out_call(s2p, dinv32, Wmu, bmu.reshape(1, 16),
                       Wls, bls.reshape(1, 16))
    return (mu, ls)


# speedup vs baseline: 53.2888x; 1.0819x over previous
import functools

import jax
import jax.numpy as jnp
from jax import lax
from jax.experimental import pallas as pl
from jax.experimental.pallas import tpu as pltpu
from jax.experimental.pallas import tpu_sc as plsc

N = 10000
E = 320000
NPAD = 10240
ROWS_PT = NPAD // 16
NGRP = ROWS_PT // 16
NC = 2
NS = 16
NW = NC * NS
EPW = E // NW
CHUNK = 80
NCH = EPW // CHUNK
HROWS = ROWS_PT // 2
NBUF = 4

_MESH = plsc.VectorSubcoreMesh(core_axis_name="c", subcore_axis_name="s")
_SC_PARAMS = pltpu.CompilerParams(use_tc_tiling_on_sc=False,
                                  needs_layout_passes=False)


def _fast_rsqrt(x):
    i = plsc.bitcast(x, jnp.int32)
    i = 0x5F3759DF - lax.shift_right_logical(i, 1)
    y = plsc.bitcast(i, jnp.float32)
    xh = 0.5 * x
    for _ in range(3):
        y = y * (1.5 - xh * y * y)
    return jnp.where(x > 0.0, y, 0.0)


def _zero_acc(zeros_hbm, acc_sh, row0):
    pltpu.sync_copy(zeros_hbm.at[pl.ds(row0, ROWS_PT)],
                    acc_sh.at[pl.ds(row0, ROWS_PT)])


def _agg_pipeline(table_hbm, sidx_v, didx_v, rows_v, acc_sh, gsem, ssem):
    def sidx(j):
        return sidx_v.at[pl.ds(j * CHUNK, CHUNK)]

    def didx(j):
        return didx_v.at[pl.ds(j * CHUNK, CHUNK)]

    for b in range(NBUF - 1):
        pltpu.async_copy(table_hbm.at[sidx(b)], rows_v.at[b], gsem.at[b])

    def body(j, carry):
        b = lax.rem(j, NBUF)
        pltpu.make_async_copy(table_hbm.at[sidx(j)], rows_v.at[b],
                              gsem.at[b]).wait()
        pltpu.async_copy(rows_v.at[b], acc_sh.at[didx(j)],
                         ssem.at[b], add=True)
        g = j + NBUF - 1
        bg = lax.rem(g, NBUF)

        @pl.when(jnp.logical_and(j >= 1, g < NCH))
        def _wait_prev_scatter():
            pltpu.make_async_copy(rows_v.at[bg], acc_sh.at[didx(0)],
                                  ssem.at[bg]).wait()

        @pl.when(g < NCH)
        def _issue_gather():
            pltpu.async_copy(table_hbm.at[sidx(g)], rows_v.at[bg],
                             gsem.at[bg])

        return carry

    lax.fori_loop(0, NCH, body, 0)
    for k in range(NBUF):
        b = (NCH - NBUF + k) % NBUF
        pltpu.make_async_copy(rows_v.at[b], acc_sh.at[didx(0)],
                              ssem.at[b]).wait()


def _sca_body(z_hbm, ei_hbm, zeros_hbm,
              s1_out, y1_out, dinv_out,
              sidx_v, didx_v, hist_v, red_v, zrow_v, dinv32_v, dinv_v,
              rows_v, hist_sh, acc_sh, gsem, ssem):
    cid = lax.axis_index("c")
    sid = lax.axis_index("s")
    wid = 2 * sid + cid
    row0 = sid * ROWS_PT

    def zero_hist(i, c):
        for u in range(8):
            hist_v[pl.ds(i * 128 + u * 16, 16)] = jnp.zeros((16,), jnp.float32)
        return c

    lax.fori_loop(0, NPAD // 128, zero_hist, 0)
    ones16 = jnp.ones((16,), jnp.float32)

    def hist_row(i, c):
        for u in range(5):
            for k in range(CHUNK // 16):
                idx16 = didx_v[pl.ds((i * 5 + u) * CHUNK + k * 16, 16)]
                plsc.addupdate_scatter(hist_v, [idx16], ones16)
        return c

    for half in range(2):
        pltpu.sync_copy(ei_hbm.at[1, pl.ds((2 * sid + half) * EPW, EPW)],
                        didx_v)
        lax.fori_loop(0, NCH // 5, hist_row, 0)
    pltpu.sync_copy(hist_v, hist_sh.at[sid])
    plsc.subcore_barrier()
    for k in range(NS):
        pltpu.sync_copy(hist_sh.at[k, pl.ds(row0, ROWS_PT)], red_v.at[k])

    def reduce_grp(g, c):
        acc = jnp.zeros((16,), jnp.float32)
        for k in range(NS):
            acc = acc + red_v[k, pl.ds(g * 16, 16)]
        dinv_v[pl.ds(g * 16, 16)] = _fast_rsqrt(acc)
        return c

    lax.fori_loop(0, NGRP, reduce_grp, 0)

    for half in range(2):
        hbase = half * HROWS

        def scale_row(i, c):
            for u in range(4):
                r = i * 4 + u
                idxr = jnp.broadcast_to(hbase + r, (16,)).astype(jnp.int32)
                dv = plsc.load_gather(dinv_v, [idxr])
                zrow_v[r, pl.ds(0, 16)] = zrow_v[r, pl.ds(0, 16)] * dv
                zrow_v[r, pl.ds(16, 16)] = zrow_v[r, pl.ds(16, 16)] * dv
                dinv32_v[r, pl.ds(0, 16)] = dv
                dinv32_v[r, pl.ds(16, 16)] = dv
            return c

        pltpu.sync_copy(z_hbm.at[pl.ds(row0 + hbase, HROWS)], zrow_v)
        lax.fori_loop(0, HROWS // 4, scale_row, 0)
        pltpu.sync_copy(
            zrow_v, y1_out.at[pl.ds(cid * NPAD + row0 + hbase, HROWS)])

        @pl.when(cid == 0)
        def _dump_dinv():
            pltpu.sync_copy(dinv32_v, dinv_out.at[pl.ds(row0 + hbase, HROWS)])

    _zero_acc(zeros_hbm, acc_sh, row0)
    pltpu.sync_copy(ei_hbm.at[0, pl.ds(wid * EPW, EPW)], sidx_v)
    pltpu.sync_copy(ei_hbm.at[1, pl.ds(wid * EPW, EPW)], didx_v)
    off16 = jnp.broadcast_to(cid * NPAD, (16,)).astype(jnp.int32)

    def offset_chunk(j, c):
        for k in range(CHUNK // 16):
            sl = pl.ds(j * CHUNK + k * 16, 16)
            sidx_v[sl] = sidx_v[sl] + off16
        return c

    lax.fori_loop(0, NCH, offset_chunk, 0)
    plsc.subcore_barrier()

    _agg_pipeline(y1_out, sidx_v, didx_v, rows_v, acc_sh, gsem, ssem)
    plsc.subcore_barrier()
    pltpu.sync_copy(acc_sh.at[pl.ds(row0, ROWS_PT)],
                    s1_out.at[cid, pl.ds(row0, ROWS_PT)])


_sca_call = functools.partial(
    pl.kernel,
    out_type=(
        jax.ShapeDtypeStruct((NC, NPAD, 32), jnp.float32),
        jax.ShapeDtypeStruct((NC * NPAD, 32), jnp.float32),
        jax.ShapeDtypeStruct((NPAD, 32), jnp.float32),
    ),
    mesh=_MESH,
    compiler_params=_SC_PARAMS,
    scratch_types=[
        pltpu.VMEM((EPW,), jnp.int32),
        pltpu.VMEM((EPW,), jnp.int32),
        pltpu.VMEM((NPAD,), jnp.float32),
        pltpu.VMEM((NS, ROWS_PT), jnp.float32),
        pltpu.VMEM((HROWS, 32), jnp.float32),
        pltpu.VMEM((HROWS, 32), jnp.float32),
        pltpu.VMEM((ROWS_PT,), jnp.float32),
        pltpu.VMEM((NBUF, CHUNK, 32), jnp.float32),
        pltpu.VMEM_SHARED((NS, NPAD), jnp.float32),
        pltpu.VMEM_SHARED((NPAD, 32), jnp.float32),
        pltpu.SemaphoreType.DMA((NBUF,)),
        pltpu.SemaphoreType.DMA((NBUF,)),
    ],
)(_sca_body)


def _scb_body(s1p_hbm, dinv_hbm, b1_hbm, ei_hbm, zeros_hbm,
              s2_out, y2_out,
              sidx_v, didx_v, s0row_v, s1row_v, dinv32_v, b1_v,
              rows_v, acc_sh, gsem, ssem):
    cid = lax.axis_index("c")
    sid = lax.axis_index("s")
    wid = 2 * sid + cid
    row0 = sid * ROWS_PT

    pltpu.sync_copy(b1_hbm, b1_v)

    def mid_row(i, c):
        for u in range(4):
            r = i * 4 + u
            for k in range(2):
                sl = pl.ds(k * 16, 16)
                s = s0row_v[r, sl] + s1row_v[r, sl]
                dv = dinv32_v[r, sl]
                h = jnp.maximum(dv * s + b1_v[sl], 0.0)
                s0row_v[r, sl] = dv * h
        return c

    for half in range(2):
        hbase = half * HROWS
        pltpu.sync_copy(s1p_hbm.at[0, pl.ds(row0 + hbase, HROWS)], s0row_v)
        pltpu.sync_copy(s1p_hbm.at[1, pl.ds(row0 + hbase, HROWS)], s1row_v)
        pltpu.sync_copy(dinv_hbm.at[pl.ds(row0 + hbase, HROWS)], dinv32_v)
        lax.fori_loop(0, HROWS // 4, mid_row, 0)
        pltpu.sync_copy(
            s0row_v, y2_out.at[pl.ds(cid * NPAD + row0 + hbase, HROWS)])

    _zero_acc(zeros_hbm, acc_sh, row0)
    pltpu.sync_copy(ei_hbm.at[0, pl.ds(wid * EPW, EPW)], sidx_v)
    pltpu.sync_copy(ei_hbm.at[1, pl.ds(wid * EPW, EPW)], didx_v)
    off16 = jnp.broadcast_to(cid * NPAD, (16,)).astype(jnp.int32)

    def offset_chunk(j, c):
        for k in range(CHUNK // 16):
            sl = pl.ds(j * CHUNK + k * 16, 16)
            sidx_v[sl] = sidx_v[sl] + off16
        return c

    lax.fori_loop(0, NCH, offset_chunk, 0)
    plsc.subcore_barrier()

    _agg_pipeline(y2_out, sidx_v, didx_v, rows_v, acc_sh, gsem, ssem)
    plsc.subcore_barrier()
    pltpu.sync_copy(acc_sh.at[pl.ds(row0, ROWS_PT)],
                    s2_out.at[cid, pl.ds(row0, ROWS_PT)])


_scb_call = functools.partial(
    pl.kernel,
    out_type=(
        jax.ShapeDtypeStruct((NC, NPAD, 32), jnp.float32),
        jax.ShapeDtypeStruct((NC * NPAD, 32), jnp.float32),
    ),
    mesh=_MESH,
    compiler_params=_SC_PARAMS,
    scratch_types=[
        pltpu.VMEM((EPW,), jnp.int32),
        pltpu.VMEM((EPW,), jnp.int32),
        pltpu.VMEM((HROWS, 32), jnp.float32),
        pltpu.VMEM((HROWS, 32), jnp.float32),
        pltpu.VMEM((HROWS, 32), jnp.float32),
        pltpu.VMEM((32,), jnp.float32),
        pltpu.VMEM((NBUF, CHUNK, 32), jnp.float32),
        pltpu.VMEM_SHARED((NPAD, 32), jnp.float32),
        pltpu.SemaphoreType.DMA((NBUF,)),
        pltpu.SemaphoreType.DMA((NBUF,)),
    ],
)(_scb_body)


def _mm_body(x_ref, w_ref, z_ref):
    z = jnp.dot(x_ref[...], w_ref[...], preferred_element_type=jnp.float32)
    z_ref[...] = jnp.concatenate(
        [z, jnp.zeros((NPAD - N, 32), jnp.float32)], axis=0)


_mm_call = pl.pallas_call(
    _mm_body,
    out_shape=jax.ShapeDtypeStruct((NPAD, 32), jnp.float32),
)


def _out_body(s2p_ref, dinv_ref, wmu_ref, bmu_ref, wls_ref, bls_ref,
              mu_ref, ls_ref):
    agg = dinv_ref[:N] * (s2p_ref[0, :N] + s2p_ref[1, :N])
    mu_ref[...] = jnp.dot(agg, wmu_ref[...],
                          preferred_element_type=jnp.float32) + bmu_ref[...]
    ls_ref[...] = jnp.dot(agg, wls_ref[...],
                          preferred_element_type=jnp.float32) + bls_ref[...]


_out_call = pl.pallas_call(
    _out_body,
    out_shape=(
        jax.ShapeDtypeStruct((N, 16), jnp.float32),
        jax.ShapeDtypeStruct((N, 16), jnp.float32),
    ),
)


def kernel(x, edge_index, W1, b1, Wmu, bmu, Wls, bls):
    ei = edge_index.astype(jnp.int32)
    zeros32 = jnp.zeros((NPAD, 32), jnp.float32)

    z = _mm_call(x, W1)
    s1p, _y1, dinv32 = _sca_call(z, ei, zeros32)
    s2p, _y2 = _scb_call(s1p, dinv32, b1, ei, zeros32)
    mu, ls = _out_call(s2p, dinv32, Wmu, bmu.reshape(1, 16),
                       Wls, bls.reshape(1, 16))
    return (mu, ls)


# speedup vs baseline: 53.7000x; 1.0077x over previous
import functools

import jax
import jax.numpy as jnp
from jax import lax
from jax.experimental import pallas as pl
from jax.experimental.pallas import tpu as pltpu
from jax.experimental.pallas import tpu_sc as plsc

N = 10000
E = 320000
NPAD = 10240
ROWS_PT = NPAD // 16
NGRP = ROWS_PT // 16
NC = 2
NS = 16
NW = NC * NS
EPW = E // NW
CHUNK = 80
NCH = EPW // CHUNK
HROWS = ROWS_PT // 2
NBUF = 4

_MESH = plsc.VectorSubcoreMesh(core_axis_name="c", subcore_axis_name="s")
_SC_PARAMS = pltpu.CompilerParams(use_tc_tiling_on_sc=False,
                                  needs_layout_passes=False)


def _fast_rsqrt(x):
    i = plsc.bitcast(x, jnp.int32)
    i = 0x5F3759DF - lax.shift_right_logical(i, 1)
    y = plsc.bitcast(i, jnp.float32)
    xh = 0.5 * x
    for _ in range(3):
        y = y * (1.5 - xh * y * y)
    return jnp.where(x > 0.0, y, 0.0)


def _zero_acc(zeros_hbm, acc_sh, row0):
    pltpu.sync_copy(zeros_hbm.at[pl.ds(row0, ROWS_PT)],
                    acc_sh.at[pl.ds(row0, ROWS_PT)])


def _agg_pipeline(table_hbm, sidx_v, didx_v, rows_v, acc_sh, gsem, ssem):
    def sidx(j):
        return sidx_v.at[pl.ds(j * CHUNK, CHUNK)]

    def didx(j):
        return didx_v.at[pl.ds(j * CHUNK, CHUNK)]

    for b in range(NBUF - 1):
        pltpu.async_copy(table_hbm.at[sidx(b)], rows_v.at[b], gsem.at[b])

    def body(j, carry):
        b = lax.rem(j, NBUF)
        pltpu.make_async_copy(table_hbm.at[sidx(j)], rows_v.at[b],
                              gsem.at[b]).wait()
        pltpu.async_copy(rows_v.at[b], acc_sh.at[didx(j)],
                         ssem.at[b], add=True)
        g = j + NBUF - 1
        bg = lax.rem(g, NBUF)

        @pl.when(jnp.logical_and(j >= 1, g < NCH))
        def _wait_prev_scatter():
            pltpu.make_async_copy(rows_v.at[bg], acc_sh.at[didx(0)],
                                  ssem.at[bg]).wait()

        @pl.when(g < NCH)
        def _issue_gather():
            pltpu.async_copy(table_hbm.at[sidx(g)], rows_v.at[bg],
                             gsem.at[bg])

        return carry

    lax.fori_loop(0, NCH, body, 0)
    for k in range(NBUF):
        b = (NCH - NBUF + k) % NBUF
        pltpu.make_async_copy(rows_v.at[b], acc_sh.at[didx(0)],
                              ssem.at[b]).wait()


def _sca_body(z_hbm, ei_hbm, zeros_hbm,
              s1_out, y1_out, dinv_out,
              sidx_v, didx_v, hist_v, red_v, zrow_v, dinv32_v, dinv_v,
              rows_v, hist_sh, acc_sh, gsem, ssem):
    cid = lax.axis_index("c")
    sid = lax.axis_index("s")
    wid = 2 * sid + cid
    row0 = sid * ROWS_PT

    def zero_hist(i, c):
        for u in range(8):
            hist_v[pl.ds(i * 128 + u * 16, 16)] = jnp.zeros((16,), jnp.float32)
        return c

    lax.fori_loop(0, NPAD // 128, zero_hist, 0)
    ones16 = jnp.ones((16,), jnp.float32)

    def hist_row(i, c):
        for u in range(5):
            for k in range(CHUNK // 16):
                idx16 = didx_v[pl.ds((i * 5 + u) * CHUNK + k * 16, 16)]
                plsc.addupdate_scatter(hist_v, [idx16], ones16)
        return c

    with jax.named_scope("p1_hist"):
        for half in range(2):
            pltpu.sync_copy(ei_hbm.at[1, pl.ds((2 * sid + half) * EPW, EPW)],
                            didx_v)
            lax.fori_loop(0, NCH // 5, hist_row, 0)
        pltpu.sync_copy(hist_v, hist_sh.at[sid])
        plsc.subcore_barrier()
        pltpu.sync_copy(hist_sh.at[:, pl.ds(row0, ROWS_PT)], red_v)

    def reduce_grp(g, c):
        acc = jnp.zeros((16,), jnp.float32)
        for k in range(NS):
            acc = acc + red_v[k, pl.ds(g * 16, 16)]
        dinv_v[pl.ds(g * 16, 16)] = _fast_rsqrt(acc)
        return c

    with jax.named_scope("p2_reduce"):
        lax.fori_loop(0, NGRP, reduce_grp, 0)

    for half in range(2):
        hbase = half * HROWS

        def scale_row(i, c):
            for u in range(4):
                r = i * 4 + u
                idxr = jnp.broadcast_to(hbase + r, (16,)).astype(jnp.int32)
                dv = plsc.load_gather(dinv_v, [idxr])
                zrow_v[r, pl.ds(0, 16)] = zrow_v[r, pl.ds(0, 16)] * dv
                zrow_v[r, pl.ds(16, 16)] = zrow_v[r, pl.ds(16, 16)] * dv
                dinv32_v[r, pl.ds(0, 16)] = dv
                dinv32_v[r, pl.ds(16, 16)] = dv
            return c

        with jax.named_scope("p2_scale"):
            pltpu.sync_copy(z_hbm.at[pl.ds(row0 + hbase, HROWS)], zrow_v)
            lax.fori_loop(0, HROWS // 4, scale_row, 0)
        pltpu.sync_copy(
            zrow_v, y1_out.at[pl.ds(cid * NPAD + row0 + hbase, HROWS)])

        @pl.when(cid == 0)
        def _dump_dinv():
            pltpu.sync_copy(dinv32_v, dinv_out.at[pl.ds(row0 + hbase, HROWS)])

    _zero_acc(zeros_hbm, acc_sh, row0)
    pltpu.sync_copy(ei_hbm.at[0, pl.ds(wid * EPW, EPW)], sidx_v)
    pltpu.sync_copy(ei_hbm.at[1, pl.ds(wid * EPW, EPW)], didx_v)
    off16 = jnp.broadcast_to(cid * NPAD, (16,)).astype(jnp.int32)

    def offset_chunk(j, c):
        for k in range(CHUNK // 16):
            sl = pl.ds(j * CHUNK + k * 16, 16)
            sidx_v[sl] = sidx_v[sl] + off16
        return c

    lax.fori_loop(0, NCH, offset_chunk, 0)
    plsc.subcore_barrier()

    with jax.named_scope("p3_agg"):
        _agg_pipeline(y1_out, sidx_v, didx_v, rows_v, acc_sh, gsem, ssem)
    plsc.subcore_barrier()
    pltpu.sync_copy(acc_sh.at[pl.ds(row0, ROWS_PT)],
                    s1_out.at[cid, pl.ds(row0, ROWS_PT)])


_sca_call = functools.partial(
    pl.kernel,
    out_type=(
        jax.ShapeDtypeStruct((NC, NPAD, 32), jnp.float32),
        jax.ShapeDtypeStruct((NC * NPAD, 32), jnp.float32),
        jax.ShapeDtypeStruct((NPAD, 32), jnp.float32),
    ),
    mesh=_MESH,
    compiler_params=_SC_PARAMS,
    scratch_types=[
        pltpu.VMEM((EPW,), jnp.int32),
        pltpu.VMEM((EPW,), jnp.int32),
        pltpu.VMEM((NPAD,), jnp.float32),
        pltpu.VMEM((NS, ROWS_PT), jnp.float32),
        pltpu.VMEM((HROWS, 32), jnp.float32),
        pltpu.VMEM((HROWS, 32), jnp.float32),
        pltpu.VMEM((ROWS_PT,), jnp.float32),
        pltpu.VMEM((NBUF, CHUNK, 32), jnp.float32),
        pltpu.VMEM_SHARED((NS, NPAD), jnp.float32),
        pltpu.VMEM_SHARED((NPAD, 32), jnp.float32),
        pltpu.SemaphoreType.DMA((NBUF,)),
        pltpu.SemaphoreType.DMA((NBUF,)),
    ],
)(_sca_body)


def _scb_body(s1p_hbm, dinv_hbm, b1_hbm, ei_hbm, zeros_hbm,
              s2_out, y2_out,
              sidx_v, didx_v, s0row_v, s1row_v, dinv32_v, b1_v,
              rows_v, acc_sh, gsem, ssem):
    cid = lax.axis_index("c")
    sid = lax.axis_index("s")
    wid = 2 * sid + cid
    row0 = sid * ROWS_PT

    pltpu.sync_copy(b1_hbm, b1_v)

    def mid_row(i, c):
        for u in range(4):
            r = i * 4 + u
            for k in range(2):
                sl = pl.ds(k * 16, 16)
                s = s0row_v[r, sl] + s1row_v[r, sl]
                dv = dinv32_v[r, sl]
                h = jnp.maximum(dv * s + b1_v[sl], 0.0)
                s0row_v[r, sl] = dv * h
        return c

    with jax.named_scope("q1_mid"):
        for half in range(2):
            hbase = half * HROWS
            pltpu.sync_copy(s1p_hbm.at[0, pl.ds(row0 + hbase, HROWS)], s0row_v)
            pltpu.sync_copy(s1p_hbm.at[1, pl.ds(row0 + hbase, HROWS)], s1row_v)
            pltpu.sync_copy(dinv_hbm.at[pl.ds(row0 + hbase, HROWS)], dinv32_v)
            lax.fori_loop(0, HROWS // 4, mid_row, 0)
            pltpu.sync_copy(
                s0row_v, y2_out.at[pl.ds(cid * NPAD + row0 + hbase, HROWS)])

    _zero_acc(zeros_hbm, acc_sh, row0)
    pltpu.sync_copy(ei_hbm.at[0, pl.ds(wid * EPW, EPW)], sidx_v)
    pltpu.sync_copy(ei_hbm.at[1, pl.ds(wid * EPW, EPW)], didx_v)
    off16 = jnp.broadcast_to(cid * NPAD, (16,)).astype(jnp.int32)

    def offset_chunk(j, c):
        for k in range(CHUNK // 16):
            sl = pl.ds(j * CHUNK + k * 16, 16)
            sidx_v[sl] = sidx_v[sl] + off16
        return c

    lax.fori_loop(0, NCH, offset_chunk, 0)
    plsc.subcore_barrier()

    with jax.named_scope("q2_agg"):
        _agg_pipeline(y2_out, sidx_v, didx_v, rows_v, acc_sh, gsem, ssem)
    plsc.subcore_barrier()
    pltpu.sync_copy(acc_sh.at[pl.ds(row0, ROWS_PT)],
                    s2_out.at[cid, pl.ds(row0, ROWS_PT)])


_scb_call = functools.partial(
    pl.kernel,
    out_type=(
        jax.ShapeDtypeStruct((NC, NPAD, 32), jnp.float32),
        jax.ShapeDtypeStruct((NC * NPAD, 32), jnp.float32),
    ),
    mesh=_MESH,
    compiler_params=_SC_PARAMS,
    scratch_types=[
        pltpu.VMEM((EPW,), jnp.int32),
        pltpu.VMEM((EPW,), jnp.int32),
        pltpu.VMEM((HROWS, 32), jnp.float32),
        pltpu.VMEM((HROWS, 32), jnp.float32),
        pltpu.VMEM((HROWS, 32), jnp.float32),
        pltpu.VMEM((32,), jnp.float32),
        pltpu.VMEM((NBUF, CHUNK, 32), jnp.float32),
        pltpu.VMEM_SHARED((NPAD, 32), jnp.float32),
        pltpu.SemaphoreType.DMA((NBUF,)),
        pltpu.SemaphoreType.DMA((NBUF,)),
    ],
)(_scb_body)


def _mm_body(x_ref, w_ref, z_ref):
    z = jnp.dot(x_ref[...], w_ref[...], preferred_element_type=jnp.float32)
    z_ref[...] = jnp.concatenate(
        [z, jnp.zeros((NPAD - N, 32), jnp.float32)], axis=0)


_mm_call = pl.pallas_call(
    _mm_body,
    out_shape=jax.ShapeDtypeStruct((NPAD, 32), jnp.float32),
)


def _out_body(s2p_ref, dinv_ref, wmu_ref, bmu_ref, wls_ref, bls_ref,
              mu_ref, ls_ref):
    agg = dinv_ref[:N] * (s2p_ref[0, :N] + s2p_ref[1, :N])
    mu_ref[...] = jnp.dot(agg, wmu_ref[...],
                          preferred_element_type=jnp.float32) + bmu_ref[...]
    ls_ref[...] = jnp.dot(agg, wls_ref[...],
                          preferred_element_type=jnp.float32) + bls_ref[...]


_out_call = pl.pallas_call(
    _out_body,
    out_shape=(
        jax.ShapeDtypeStruct((N, 16), jnp.float32),
        jax.ShapeDtypeStruct((N, 16), jnp.float32),
    ),
)


def kernel(x, edge_index, W1, b1, Wmu, bmu, Wls, bls):
    ei = edge_index.astype(jnp.int32)
    zeros32 = jnp.zeros((NPAD, 32), jnp.float32)

    z = _mm_call(x, W1)
    s1p, _y1, dinv32 = _sca_call(z, ei, zeros32)
    s2p, _y2 = _scb_call(s1p, dinv32, b1, ei, zeros32)
    mu, ls = _out_call(s2p, dinv32, Wmu, bmu.reshape(1, 16),
                       Wls, bls.reshape(1, 16))
    return (mu, ls)


# speedup vs baseline: 69.5387x; 1.2949x over previous
import functools

import jax
import jax.numpy as jnp
from jax import lax
from jax.experimental import pallas as pl
from jax.experimental.pallas import tpu as pltpu
from jax.experimental.pallas import tpu_sc as plsc

N = 10000
E = 320000
NPAD = 10240
ROWS_PT = NPAD // 16
NGRP = ROWS_PT // 16
NC = 2
NS = 16
NW = NC * NS
EPW = E // NW
CHUNK = 128
NCHF = EPW // CHUNK
TAIL = EPW - NCHF * CHUNK
HCH = 80
NHC = EPW // HCH
HROWS = ROWS_PT // 2
NBUF = 6

_MESH = plsc.VectorSubcoreMesh(core_axis_name="c", subcore_axis_name="s")
_SC_PARAMS = pltpu.CompilerParams(use_tc_tiling_on_sc=False,
                                  needs_layout_passes=False)


def _fast_rsqrt(x):
    i = plsc.bitcast(x, jnp.int32)
    i = 0x5F3759DF - lax.shift_right_logical(i, 1)
    y = plsc.bitcast(i, jnp.float32)
    xh = 0.5 * x
    for _ in range(3):
        y = y * (1.5 - xh * y * y)
    return jnp.where(x > 0.0, y, 0.0)


def _zero_acc(zeros_hbm, acc_sh, row0):
    pltpu.sync_copy(zeros_hbm.at[pl.ds(row0, ROWS_PT)],
                    acc_sh.at[pl.ds(row0, ROWS_PT)])


def _agg_pipeline(table_hbm, sidx_v, didx_v, rows_v, acc_sh, gsem, ssem):
    def sidx(j):
        return sidx_v.at[pl.ds(j * CHUNK, CHUNK)]

    def didx(j):
        return didx_v.at[pl.ds(j * CHUNK, CHUNK)]

    for b in range(NBUF - 1):
        pltpu.async_copy(table_hbm.at[sidx(b)], rows_v.at[b], gsem.at[b])

    def body(j, carry):
        b = lax.rem(j, NBUF)
        pltpu.make_async_copy(table_hbm.at[sidx(j)], rows_v.at[b],
                              gsem.at[b]).wait()
        pltpu.async_copy(rows_v.at[b], acc_sh.at[didx(j)],
                         ssem.at[b], add=True)
        g = j + NBUF - 1
        bg = lax.rem(g, NBUF)

        @pl.when(jnp.logical_and(j >= 1, g < NCHF))
        def _wait_prev_scatter():
            pltpu.make_async_copy(rows_v.at[bg], acc_sh.at[didx(0)],
                                  ssem.at[bg]).wait()

        @pl.when(g < NCHF)
        def _issue_gather():
            pltpu.async_copy(table_hbm.at[sidx(g)], rows_v.at[bg],
                             gsem.at[bg])

        return carry

    lax.fori_loop(0, NCHF, body, 0)
    for k in range(NBUF):
        b = (NCHF - NBUF + k) % NBUF
        pltpu.make_async_copy(rows_v.at[b], acc_sh.at[didx(0)],
                              ssem.at[b]).wait()
    toff = NCHF * CHUNK
    pltpu.async_copy(table_hbm.at[sidx_v.at[pl.ds(toff, TAIL)]],
                     rows_v.at[0, pl.ds(0, TAIL)], gsem.at[0])
    pltpu.make_async_copy(table_hbm.at[sidx_v.at[pl.ds(toff, TAIL)]],
                          rows_v.at[0, pl.ds(0, TAIL)], gsem.at[0]).wait()
    pltpu.sync_copy(rows_v.at[0, pl.ds(0, TAIL)],
                    acc_sh.at[didx_v.at[pl.ds(toff, TAIL)]], add=True)


def _sca_body(z_hbm, ei_hbm, zeros_hbm,
              s1_out, y1_out, dinv_out,
              sidx_v, didx_v, hist_v, red_v, zrow_v, dinv32_v, dinv_v,
              rows_v, hist_sh, acc_sh, gsem, ssem):
    cid = lax.axis_index("c")
    sid = lax.axis_index("s")
    wid = 2 * sid + cid
    row0 = sid * ROWS_PT

    def zero_hist(i, c):
        for u in range(8):
            hist_v[pl.ds(i * 128 + u * 16, 16)] = jnp.zeros((16,), jnp.float32)
        return c

    lax.fori_loop(0, NPAD // 128, zero_hist, 0)
    ones16 = jnp.ones((16,), jnp.float32)

    def mk_hist_row(buf):
        def hist_row(i, c):
            for u in range(5):
                for k in range(HCH // 16):
                    idx16 = buf[pl.ds((i * 5 + u) * HCH + k * 16, 16)]
                    plsc.addupdate_scatter(hist_v, [idx16], ones16)
            return c
        return hist_row

    with jax.named_scope("p1_hist"):
        pltpu.async_copy(ei_hbm.at[1, pl.ds(2 * sid * EPW, EPW)],
                         didx_v, gsem.at[0])
        pltpu.async_copy(ei_hbm.at[1, pl.ds((2 * sid + 1) * EPW, EPW)],
                         sidx_v, gsem.at[1])
        pltpu.async_copy(z_hbm.at[pl.ds(row0, HROWS), pl.ds(0, 32)],
                         zrow_v, gsem.at[2])
        pltpu.make_async_copy(ei_hbm.at[1, pl.ds(2 * sid * EPW, EPW)],
                              didx_v, gsem.at[0]).wait()
        lax.fori_loop(0, NHC // 5, mk_hist_row(didx_v), 0)
        pltpu.make_async_copy(ei_hbm.at[1, pl.ds(2 * sid * EPW, EPW)],
                              sidx_v, gsem.at[1]).wait()
        lax.fori_loop(0, NHC // 5, mk_hist_row(sidx_v), 0)
        pltpu.sync_copy(hist_v, hist_sh.at[sid])
        plsc.subcore_barrier()
        pltpu.sync_copy(hist_sh.at[:, pl.ds(row0, ROWS_PT)], red_v)

    def reduce_grp(g, c):
        acc = jnp.zeros((16,), jnp.float32)
        for k in range(NS):
            acc = acc + red_v[k, pl.ds(g * 16, 16)]
        dinv_v[pl.ds(g * 16, 16)] = _fast_rsqrt(acc)
        return c

    with jax.named_scope("p2_reduce"):
        lax.fori_loop(0, NGRP, reduce_grp, 0)

    for half in range(2):
        hbase = half * HROWS

        def scale_row(i, c):
            for u in range(4):
                r = i * 4 + u
                idxr = jnp.broadcast_to(hbase + r, (16,)).astype(jnp.int32)
                dv = plsc.load_gather(dinv_v, [idxr])
                zrow_v[r, pl.ds(0, 16)] = zrow_v[r, pl.ds(0, 16)] * dv
                zrow_v[r, pl.ds(16, 16)] = zrow_v[r, pl.ds(16, 16)] * dv
                dinv32_v[r, pl.ds(0, 16)] = dv
                dinv32_v[r, pl.ds(16, 16)] = dv
            return c

        with jax.named_scope("p2_scale"):
            pltpu.make_async_copy(
                z_hbm.at[pl.ds(row0 + hbase, HROWS), pl.ds(0, 32)],
                zrow_v, gsem.at[2]).wait()
            lax.fori_loop(0, HROWS // 4, scale_row, 0)
        pltpu.sync_copy(
            zrow_v, y1_out.at[pl.ds(cid * NPAD + row0 + hbase, HROWS)])
        if half == 0:
            pltpu.async_copy(z_hbm.at[pl.ds(row0 + HROWS, HROWS), pl.ds(0, 32)],
                             zrow_v, gsem.at[2])

        @pl.when(cid == 0)
        def _dump_dinv():
            pltpu.sync_copy(dinv32_v, dinv_out.at[pl.ds(row0 + hbase, HROWS),
                                                  pl.ds(0, 32)])

    _zero_acc(zeros_hbm, acc_sh, row0)
    pltpu.sync_copy(ei_hbm.at[0, pl.ds(wid * EPW, EPW)], sidx_v)
    pltpu.sync_copy(ei_hbm.at[1, pl.ds(wid * EPW, EPW)], didx_v)
    off16 = jnp.broadcast_to(cid * NPAD, (16,)).astype(jnp.int32)

    def offset_chunk(j, c):
        for k in range(HCH // 16):
            sl = pl.ds(j * HCH + k * 16, 16)
            sidx_v[sl] = sidx_v[sl] + off16
        return c

    lax.fori_loop(0, NHC, offset_chunk, 0)
    plsc.subcore_barrier()

    with jax.named_scope("p3_agg"):
        _agg_pipeline(y1_out, sidx_v, didx_v, rows_v, acc_sh, gsem, ssem)
    plsc.subcore_barrier()
    pltpu.sync_copy(acc_sh.at[pl.ds(row0, ROWS_PT)],
                    s1_out.at[pl.ds(row0, ROWS_PT), pl.ds(cid * 32, 32)])


_sca_call = functools.partial(
    pl.kernel,
    out_type=(
        jax.ShapeDtypeStruct((NPAD, 128), jnp.float32),
        jax.ShapeDtypeStruct((NC * NPAD, 32), jnp.float32),
        jax.ShapeDtypeStruct((NPAD, 128), jnp.float32),
    ),
    mesh=_MESH,
    compiler_params=_SC_PARAMS,
    scratch_types=[
        pltpu.VMEM((EPW,), jnp.int32),
        pltpu.VMEM((EPW,), jnp.int32),
        pltpu.VMEM((NPAD,), jnp.float32),
        pltpu.VMEM((NS, ROWS_PT), jnp.float32),
        pltpu.VMEM((HROWS, 32), jnp.float32),
        pltpu.VMEM((HROWS, 32), jnp.float32),
        pltpu.VMEM((ROWS_PT,), jnp.float32),
        pltpu.VMEM((NBUF, CHUNK, 32), jnp.float32),
        pltpu.VMEM_SHARED((NS, NPAD), jnp.float32),
        pltpu.VMEM_SHARED((NPAD, 32), jnp.float32),
        pltpu.SemaphoreType.DMA((NBUF,)),
        pltpu.SemaphoreType.DMA((NBUF,)),
    ],
)(_sca_body)


def _scb_body(s1p_hbm, dinv_hbm, b1_hbm, ei_hbm, zeros_hbm,
              s2_out, y2_out,
              sidx_v, didx_v, s0row_v, s1row_v, dinv32_v, b1_v,
              rows_v, acc_sh, gsem, ssem):
    cid = lax.axis_index("c")
    sid = lax.axis_index("s")
    wid = 2 * sid + cid
    row0 = sid * ROWS_PT

    pltpu.sync_copy(b1_hbm, b1_v)

    def mid_row(i, c):
        for u in range(4):
            r = i * 4 + u
            for k in range(2):
                sl = pl.ds(k * 16, 16)
                s = s0row_v[r, sl] + s1row_v[r, sl]
                dv = dinv32_v[r, sl]
                h = jnp.maximum(dv * s + b1_v[sl], 0.0)
                s0row_v[r, sl] = dv * h
        return c

    with jax.named_scope("q1_mid"):
        pltpu.async_copy(ei_hbm.at[0, pl.ds(wid * EPW, EPW)], sidx_v,
                         gsem.at[3])
        pltpu.async_copy(ei_hbm.at[1, pl.ds(wid * EPW, EPW)], didx_v,
                         gsem.at[4])
        for half in range(2):
            hbase = half * HROWS
            rsl = pl.ds(row0 + hbase, HROWS)
            pltpu.async_copy(s1p_hbm.at[rsl, pl.ds(0, 32)],
                             s0row_v, gsem.at[0])
            pltpu.async_copy(s1p_hbm.at[rsl, pl.ds(32, 32)],
                             s1row_v, gsem.at[1])
            pltpu.async_copy(dinv_hbm.at[rsl, pl.ds(0, 32)],
                             dinv32_v, gsem.at[2])
            pltpu.make_async_copy(s1p_hbm.at[rsl, pl.ds(0, 32)],
                                  s0row_v, gsem.at[0]).wait()
            pltpu.make_async_copy(s1p_hbm.at[rsl, pl.ds(32, 32)],
                                  s1row_v, gsem.at[1]).wait()
            pltpu.make_async_copy(dinv_hbm.at[rsl, pl.ds(0, 32)],
                                  dinv32_v, gsem.at[2]).wait()
            lax.fori_loop(0, HROWS // 4, mid_row, 0)
            pltpu.sync_copy(
                s0row_v, y2_out.at[pl.ds(cid * NPAD + row0 + hbase, HROWS)])

    _zero_acc(zeros_hbm, acc_sh, row0)
    pltpu.make_async_copy(ei_hbm.at[0, pl.ds(wid * EPW, EPW)], sidx_v,
                          gsem.at[3]).wait()
    pltpu.make_async_copy(ei_hbm.at[1, pl.ds(wid * EPW, EPW)], didx_v,
                          gsem.at[4]).wait()
    off16 = jnp.broadcast_to(cid * NPAD, (16,)).astype(jnp.int32)

    def offset_chunk(j, c):
        for k in range(HCH // 16):
            sl = pl.ds(j * HCH + k * 16, 16)
            sidx_v[sl] = sidx_v[sl] + off16
        return c

    lax.fori_loop(0, NHC, offset_chunk, 0)
    plsc.subcore_barrier()

    with jax.named_scope("q2_agg"):
        _agg_pipeline(y2_out, sidx_v, didx_v, rows_v, acc_sh, gsem, ssem)
    plsc.subcore_barrier()
    pltpu.sync_copy(acc_sh.at[pl.ds(row0, ROWS_PT)],
                    s2_out.at[pl.ds(row0, ROWS_PT), pl.ds(cid * 32, 32)])


_scb_call = functools.partial(
    pl.kernel,
    out_type=(
        jax.ShapeDtypeStruct((NPAD, 128), jnp.float32),
        jax.ShapeDtypeStruct((NC * NPAD, 32), jnp.float32),
    ),
    mesh=_MESH,
    compiler_params=_SC_PARAMS,
    scratch_types=[
        pltpu.VMEM((EPW,), jnp.int32),
        pltpu.VMEM((EPW,), jnp.int32),
        pltpu.VMEM((HROWS, 32), jnp.float32),
        pltpu.VMEM((HROWS, 32), jnp.float32),
        pltpu.VMEM((HROWS, 32), jnp.float32),
        pltpu.VMEM((32,), jnp.float32),
        pltpu.VMEM((NBUF, CHUNK, 32), jnp.float32),
        pltpu.VMEM_SHARED((NPAD, 32), jnp.float32),
        pltpu.SemaphoreType.DMA((NBUF,)),
        pltpu.SemaphoreType.DMA((NBUF,)),
    ],
)(_scb_body)


def _mm_body(x_ref, w_ref, z_ref):
    z = jnp.dot(x_ref[...], w_ref[...], preferred_element_type=jnp.float32)
    z = jnp.concatenate([z, jnp.zeros((NPAD - N, 32), jnp.float32)], axis=0)
    z_ref[...] = jnp.concatenate(
        [z, jnp.zeros((NPAD, 96), jnp.float32)], axis=1)


_mm_call = pl.pallas_call(
    _mm_body,
    out_shape=jax.ShapeDtypeStruct((NPAD, 128), jnp.float32),
)


def _out_body(s2p_ref, dinv_ref, wmu_ref, bmu_ref, wls_ref, bls_ref,
              mu_ref, ls_ref):
    s2p = s2p_ref[:N]
    agg = dinv_ref[:N, 0:32] * (s2p[:, 0:32] + s2p[:, 32:64])
    mu_ref[...] = jnp.dot(agg, wmu_ref[...],
                          preferred_element_type=jnp.float32) + bmu_ref[...]
    ls_ref[...] = jnp.dot(agg, wls_ref[...],
                          preferred_element_type=jnp.float32) + bls_ref[...]


_out_call = pl.pallas_call(
    _out_body,
    out_shape=(
        jax.ShapeDtypeStruct((N, 16), jnp.float32),
        jax.ShapeDtypeStruct((N, 16), jnp.float32),
    ),
)


def kernel(x, edge_index, W1, b1, Wmu, bmu, Wls, bls):
    ei = edge_index.astype(jnp.int32)
    zeros32 = jnp.zeros((NPAD, 32), jnp.float32)

    z = _mm_call(x, W1)
    s1p, _y1, dinv32 = _sca_call(z, ei, zeros32)
    s2p, _y2 = _scb_call(s1p, dinv32, b1, ei, zeros32)
    mu, ls = _out_call(s2p, dinv32, Wmu, bmu.reshape(1, 16),
                       Wls, bls.reshape(1, 16))
    return (mu, ls)
